# Initial kernel scaffold; baseline (speedup 1.0000x reference)
import functools

import jax
import jax.numpy as jnp
from jax import lax
from jax.experimental import pallas as pl
from jax.experimental.pallas import tpu as pltpu
from jax.experimental.pallas import tpu_sc as plsc

_N_EMB = 8192
_DIM = 256
_N_TOK = 8192

_BT = 1024
_BK = 1024
_NTB = _N_TOK // _BT
_NKB = _N_EMB // _BK


def _argmin_body(z_ref, emb_ref, zsq_ref, esq_ref, idx_ref, dsum_ref,
                 best_ref, barg_ref):
    kb = pl.program_id(1)
    ez = lax.dot_general(z_ref[...], emb_ref[...], (((1,), (1,)), ((), ())),
                         preferred_element_type=jnp.float32)
    d = (zsq_ref[...] + esq_ref[...]) - 2.0 * ez
    m = jnp.min(d, axis=1, keepdims=True)
    lane = lax.broadcasted_iota(jnp.int32, (_BT, _BK), 1)
    a = jnp.min(jnp.where(d == m, lane, _BK), axis=1, keepdims=True) + kb * _BK
    upd = jnp.logical_or(kb == 0, m < best_ref[...])
    best_ref[...] = jnp.where(upd, m, best_ref[...])
    barg_ref[...] = jnp.where(upd, a, barg_ref[...])

    @pl.when(kb == _NKB - 1)
    def _():
        idx_ref[...] = barg_ref[...]
        dsum_ref[0, 0, 0] = jnp.sum(best_ref[...])


def _tc_argmin(z_flat, emb, zsq, esq):
    return pl.pallas_call(
        _argmin_body,
        grid=(_NTB, _NKB),
        in_specs=[
            pl.BlockSpec((_BT, _DIM), lambda tb, kb: (tb, 0)),
            pl.BlockSpec((_BK, _DIM), lambda tb, kb: (kb, 0)),
            pl.BlockSpec((_BT, 1), lambda tb, kb: (tb, 0)),
            pl.BlockSpec((1, _BK), lambda tb, kb: (0, kb)),
        ],
        out_specs=[
            pl.BlockSpec((_BT, 1), lambda tb, kb: (tb, 0)),
            pl.BlockSpec((1, 1, 1), lambda tb, kb: (tb, 0, 0)),
        ],
        out_shape=[
            jax.ShapeDtypeStruct((_N_TOK, 1), jnp.int32),
            jax.ShapeDtypeStruct((_NTB, 1, 1), jnp.float32),
        ],
        scratch_shapes=[
            pltpu.VMEM((_BT, 1), jnp.float32),
            pltpu.VMEM((_BT, 1), jnp.int32),
        ],
    )(z_flat, emb, zsq, esq)


_SC_NC = 2
_SC_NS = 16
_SC_NW = _SC_NC * _SC_NS
_ROWS_PER_W = _N_TOK // _SC_NW


@functools.cache
def _sc_gather_kernel():
    mesh = plsc.VectorSubcoreMesh(core_axis_name="c", subcore_axis_name="s")

    @functools.partial(
        pl.kernel,
        out_type=jax.ShapeDtypeStruct((_N_TOK, _DIM), jnp.float32),
        mesh=mesh,
        scratch_types=[
            pltpu.VMEM((_ROWS_PER_W,), jnp.int32),
            pltpu.VMEM((_ROWS_PER_W, _DIM), jnp.float32),
            pltpu.SemaphoreType.DMA,
        ],
    )
    def gather(emb_hbm, idx_hbm, out_hbm, idx_v, rows_v, sem):
        wid = lax.axis_index("s") * _SC_NC + lax.axis_index("c")
        base = wid * _ROWS_PER_W
        pltpu.sync_copy(idx_hbm.at[pl.ds(base, _ROWS_PER_W)], idx_v)
        pltpu.async_copy(emb_hbm.at[idx_v], rows_v, sem).wait()
        pltpu.sync_copy(rows_v, out_hbm.at[pl.ds(base, _ROWS_PER_W)])

    return gather


def kernel(z, emb):
    bs, dim, h, w = z.shape
    zp = jnp.transpose(z, (0, 2, 3, 1))
    z_flat = zp.reshape(-1, dim)
    zsq = jnp.sum(z_flat ** 2, axis=1, keepdims=True)
    esq = jnp.sum(emb ** 2, axis=1)[None, :]
    idx, dsum = _tc_argmin(z_flat, emb, zsq, esq)
    z_q = _sc_gather_kernel()(emb, idx.reshape(-1))
    loss = jnp.sum(dsum) * (1.25 / (_N_TOK * _DIM))
    z_q_out = jnp.transpose(z_q.reshape(bs, h, w, dim), (0, 3, 1, 2))
    return (z_q_out, loss)


# speedup vs baseline: 1.1369x; 1.1369x over previous
import functools

import jax
import jax.numpy as jnp
from jax import lax
from jax.experimental import pallas as pl
from jax.experimental.pallas import tpu as pltpu
from jax.experimental.pallas import tpu_sc as plsc

_N_EMB = 8192
_DIM = 256
_N_TOK = 8192

_BT = 256
_NTB = _N_TOK // _BT
_BOUNDS = (0, 2048, 4096, 6144, 8192)


def _rtne_bf16(x):
    u = lax.bitcast_convert_type(x, jnp.uint32)
    odd = lax.shift_right_logical(u, jnp.uint32(16)) & jnp.uint32(1)
    u = (u + jnp.uint32(0x7FFF) + odd) & jnp.uint32(0xFFFF0000)
    return lax.bitcast_convert_type(u, jnp.float32)


def _argmin_body(z_ref, emb_ref, zsq_ref, esq_ref, idx_ref, dsum_ref):
    ez = lax.dot_general(z_ref[...], emb_ref[...], (((1,), (1,)), ((), ())),
                         preferred_element_type=jnp.float32)
    d = (zsq_ref[...] + esq_ref[...]) - 2.0 * ez
    accv = None
    acci = None
    for s in range(len(_BOUNDS) - 1):
        c0, c1 = _BOUNDS[s], _BOUNDS[s + 1]
        ds = d[:, c0:c1]
        cols = lax.broadcasted_iota(jnp.int32, (_BT, c1 - c0), 1) + c0
        m = jnp.min(ds, axis=1, keepdims=True)
        i = jnp.min(jnp.where(ds == m, cols, jnp.int32(2**30)),
                    axis=1, keepdims=True)
        if accv is None:
            accv, acci = _rtne_bf16(m), i
        else:
            win = m < accv
            accv = _rtne_bf16(jnp.where(win, m, accv))
            acci = jnp.where(win, i, acci)
    idx_ref[...] = acci
    dsum_ref[...] = jnp.sum(accv, keepdims=True).reshape(1, 1, 1)


def _tc_argmin(z_bf, emb_bf, zsq, esq):
    return pl.pallas_call(
        _argmin_body,
        grid=(_NTB,),
        in_specs=[
            pl.BlockSpec((_BT, _DIM), lambda tb: (tb, 0)),
            pl.BlockSpec((_N_EMB, _DIM), lambda tb: (0, 0)),
            pl.BlockSpec((_BT, 1), lambda tb: (tb, 0)),
            pl.BlockSpec((1, _N_EMB), lambda tb: (0, 0)),
        ],
        out_specs=[
            pl.BlockSpec((_BT, 1), lambda tb: (tb, 0)),
            pl.BlockSpec((1, 1, 1), lambda tb: (tb, 0, 0)),
        ],
        out_shape=[
            jax.ShapeDtypeStruct((_N_TOK, 1), jnp.int32),
            jax.ShapeDtypeStruct((_NTB, 1, 1), jnp.float32),
        ],
    )(z_bf, emb_bf, zsq, esq)


_SC_NC = 2
_SC_NS = 16
_SC_NW = _SC_NC * _SC_NS
_ROWS_PER_W = _N_TOK // _SC_NW


@functools.cache
def _sc_gather_kernel():
    mesh = plsc.VectorSubcoreMesh(core_axis_name="c", subcore_axis_name="s")

    @functools.partial(
        pl.kernel,
        out_type=jax.ShapeDtypeStruct((_N_TOK, _DIM), jnp.float32),
        mesh=mesh,
        scratch_types=[
            pltpu.VMEM((_ROWS_PER_W,), jnp.int32),
            pltpu.VMEM((_ROWS_PER_W, _DIM), jnp.float32),
            pltpu.SemaphoreType.DMA,
        ],
    )
    def gather(emb_hbm, idx_hbm, out_hbm, idx_v, rows_v, sem):
        wid = lax.axis_index("s") * _SC_NC + lax.axis_index("c")
        base = wid * _ROWS_PER_W
        pltpu.sync_copy(idx_hbm.at[pl.ds(base, _ROWS_PER_W)], idx_v)
        pltpu.async_copy(emb_hbm.at[idx_v], rows_v, sem).wait()
        pltpu.sync_copy(rows_v, out_hbm.at[pl.ds(base, _ROWS_PER_W)])

    return gather


def kernel(z, emb):
    bs, dim, h, w = z.shape
    zp = jnp.transpose(z, (0, 2, 3, 1))
    z_flat = zp.reshape(-1, dim)
    zsq = jnp.sum(z_flat ** 2, axis=1, keepdims=True)
    esq = jnp.sum(emb ** 2, axis=1)[None, :]
    z_bf = z_flat.astype(jnp.bfloat16)
    emb_bf = emb.astype(jnp.bfloat16)
    idx, dsum = _tc_argmin(z_bf, emb_bf, zsq, esq)
    emb_r = emb_bf.astype(jnp.float32)
    z_q = _sc_gather_kernel()(emb_r, idx.reshape(-1))
    loss = jnp.sum(dsum) * (1.25 / (_N_TOK * _DIM))
    z_q_out = jnp.transpose(z_q.reshape(bs, h, w, dim), (0, 3, 1, 2))
    return (z_q_out, loss)


# speedup vs baseline: 1.2526x; 1.1018x over previous
import functools

import jax
import jax.numpy as jnp
from jax import lax
from jax.experimental import pallas as pl
from jax.experimental.pallas import tpu as pltpu
from jax.experimental.pallas import tpu_sc as plsc

_N_EMB = 8192
_DIM = 256
_N_TOK = 8192

_BT = 512
_NTB = _N_TOK // _BT
_BOUNDS = (0, 2048, 4096, 6144, 8192)


def _rtne_bf16(x):
    u = lax.bitcast_convert_type(x, jnp.uint32)
    odd = lax.shift_right_logical(u, jnp.uint32(16)) & jnp.uint32(1)
    u = (u + jnp.uint32(0x7FFF) + odd) & jnp.uint32(0xFFFF0000)
    return lax.bitcast_convert_type(u, jnp.float32)


def _argmin_body(z_ref, emb_ref, zsq_ref, esq_ref, idx_ref, dsum_ref):
    ez = lax.dot_general(z_ref[...], emb_ref[...], (((1,), (1,)), ((), ())),
                         preferred_element_type=jnp.float32)
    d = (zsq_ref[...] + esq_ref[...]) - 2.0 * ez
    accv = None
    acci = None
    for s in range(len(_BOUNDS) - 1):
        c0, c1 = _BOUNDS[s], _BOUNDS[s + 1]
        ds = d[:, c0:c1]
        cols = lax.broadcasted_iota(jnp.int32, (_BT, c1 - c0), 1) + c0
        m = jnp.min(ds, axis=1, keepdims=True)
        i = jnp.min(jnp.where(ds == m, cols, jnp.int32(2**30)),
                    axis=1, keepdims=True)
        if accv is None:
            accv, acci = _rtne_bf16(m), i
        else:
            win = m < accv
            accv = _rtne_bf16(jnp.where(win, m, accv))
            acci = jnp.where(win, i, acci)
    idx_ref[...] = acci
    dsum_ref[...] = jnp.sum(accv, keepdims=True).reshape(1, 1, 1)


def _tc_argmin(z_bf, emb_bf, zsq, esq):
    return pl.pallas_call(
        _argmin_body,
        grid=(_NTB,),
        in_specs=[
            pl.BlockSpec((_BT, _DIM), lambda tb: (tb, 0)),
            pl.BlockSpec((_N_EMB, _DIM), lambda tb: (0, 0)),
            pl.BlockSpec((_BT, 1), lambda tb: (tb, 0)),
            pl.BlockSpec((1, _N_EMB), lambda tb: (0, 0)),
        ],
        out_specs=[
            pl.BlockSpec((_BT, 1), lambda tb: (tb, 0)),
            pl.BlockSpec((1, 1, 1), lambda tb: (tb, 0, 0)),
        ],
        out_shape=[
            jax.ShapeDtypeStruct((_N_TOK, 1), jnp.int32),
            jax.ShapeDtypeStruct((_NTB, 1, 1), jnp.float32),
        ],
    )(z_bf, emb_bf, zsq, esq)


_SC_NC = 2
_SC_NS = 16
_SC_NW = _SC_NC * _SC_NS
_ROWS_PER_W = _N_TOK // _SC_NW


@functools.cache
def _sc_gather_kernel():
    mesh = plsc.VectorSubcoreMesh(core_axis_name="c", subcore_axis_name="s")

    @functools.partial(
        pl.kernel,
        out_type=jax.ShapeDtypeStruct((_N_TOK, _DIM), jnp.float32),
        mesh=mesh,
        scratch_types=[
            pltpu.VMEM((_ROWS_PER_W,), jnp.int32),
            pltpu.VMEM((_ROWS_PER_W, _DIM), jnp.float32),
            pltpu.SemaphoreType.DMA,
        ],
    )
    def gather(emb_hbm, idx_hbm, out_hbm, idx_v, rows_v, sem):
        wid = lax.axis_index("s") * _SC_NC + lax.axis_index("c")
        base = wid * _ROWS_PER_W
        pltpu.sync_copy(idx_hbm.at[pl.ds(base, _ROWS_PER_W)], idx_v)
        pltpu.async_copy(emb_hbm.at[idx_v], rows_v, sem).wait()
        pltpu.sync_copy(rows_v, out_hbm.at[pl.ds(base, _ROWS_PER_W)])

    return gather


def kernel(z, emb):
    bs, dim, h, w = z.shape
    zp = jnp.transpose(z, (0, 2, 3, 1))
    z_flat = zp.reshape(-1, dim)
    zsq = jnp.sum(z_flat ** 2, axis=1, keepdims=True)
    esq = jnp.sum(emb ** 2, axis=1)[None, :]
    z_bf = z_flat.astype(jnp.bfloat16)
    emb_bf = emb.astype(jnp.bfloat16)
    idx, dsum = _tc_argmin(z_bf, emb_bf, zsq, esq)
    emb_r = emb_bf.astype(jnp.float32)
    z_q = _sc_gather_kernel()(emb_r, idx.reshape(-1))
    loss = jnp.sum(dsum) * (1.25 / (_N_TOK * _DIM))
    z_q_out = jnp.transpose(z_q.reshape(bs, h, w, dim), (0, 3, 1, 2))
    return (z_q_out, loss)


# speedup vs baseline: 1.3164x; 1.0509x over previous
import functools

import jax
import jax.numpy as jnp
from jax import lax
from jax.experimental import pallas as pl
from jax.experimental.pallas import tpu as pltpu
from jax.experimental.pallas import tpu_sc as plsc

_N_EMB = 8192
_DIM = 256
_N_TOK = 8192

_BT = 1024
_NTB = _N_TOK // _BT
_BOUNDS = (0, 2048, 4096, 6144, 8192)


def _rtne_bf16(x):
    u = lax.bitcast_convert_type(x, jnp.uint32)
    odd = lax.shift_right_logical(u, jnp.uint32(16)) & jnp.uint32(1)
    u = (u + jnp.uint32(0x7FFF) + odd) & jnp.uint32(0xFFFF0000)
    return lax.bitcast_convert_type(u, jnp.float32)


def _argmin_body(z_ref, emb_ref, zsq_ref, esq_ref, idx_ref, dsum_ref):
    ez = lax.dot_general(z_ref[...], emb_ref[...], (((1,), (1,)), ((), ())),
                         preferred_element_type=jnp.float32)
    d = (zsq_ref[...] + esq_ref[...]) - 2.0 * ez
    accv = None
    acci = None
    for s in range(len(_BOUNDS) - 1):
        c0, c1 = _BOUNDS[s], _BOUNDS[s + 1]
        ds = d[:, c0:c1]
        cols = lax.broadcasted_iota(jnp.int32, (_BT, c1 - c0), 1) + c0
        m = jnp.min(ds, axis=1, keepdims=True)
        i = jnp.min(jnp.where(ds == m, cols, jnp.int32(2**30)),
                    axis=1, keepdims=True)
        if accv is None:
            accv, acci = _rtne_bf16(m), i
        else:
            win = m < accv
            accv = _rtne_bf16(jnp.where(win, m, accv))
            acci = jnp.where(win, i, acci)
    idx_ref[...] = acci
    dsum_ref[...] = jnp.sum(accv, keepdims=True).reshape(1, 1, 1)


def _tc_argmin(z_bf, emb_bf, zsq, esq):
    return pl.pallas_call(
        _argmin_body,
        grid=(_NTB,),
        in_specs=[
            pl.BlockSpec((_BT, _DIM), lambda tb: (tb, 0)),
            pl.BlockSpec((_N_EMB, _DIM), lambda tb: (0, 0)),
            pl.BlockSpec((_BT, 1), lambda tb: (tb, 0)),
            pl.BlockSpec((1, _N_EMB), lambda tb: (0, 0)),
        ],
        out_specs=[
            pl.BlockSpec((_BT, 1), lambda tb: (tb, 0)),
            pl.BlockSpec((1, 1, 1), lambda tb: (tb, 0, 0)),
        ],
        out_shape=[
            jax.ShapeDtypeStruct((_N_TOK, 1), jnp.int32),
            jax.ShapeDtypeStruct((_NTB, 1, 1), jnp.float32),
        ],
    )(z_bf, emb_bf, zsq, esq)


_SC_NC = 2
_SC_NS = 16
_SC_NW = _SC_NC * _SC_NS
_ROWS_PER_W = _N_TOK // _SC_NW


@functools.cache
def _sc_gather_kernel():
    mesh = plsc.VectorSubcoreMesh(core_axis_name="c", subcore_axis_name="s")

    @functools.partial(
        pl.kernel,
        out_type=jax.ShapeDtypeStruct((_N_TOK, _DIM), jnp.float32),
        mesh=mesh,
        scratch_types=[
            pltpu.VMEM((_ROWS_PER_W,), jnp.int32),
            pltpu.VMEM((_ROWS_PER_W, _DIM), jnp.float32),
            pltpu.SemaphoreType.DMA,
        ],
    )
    def gather(emb_hbm, idx_hbm, out_hbm, idx_v, rows_v, sem):
        wid = lax.axis_index("s") * _SC_NC + lax.axis_index("c")
        base = wid * _ROWS_PER_W
        pltpu.sync_copy(idx_hbm.at[pl.ds(base, _ROWS_PER_W)], idx_v)
        pltpu.async_copy(emb_hbm.at[idx_v], rows_v, sem).wait()
        pltpu.sync_copy(rows_v, out_hbm.at[pl.ds(base, _ROWS_PER_W)])

    return gather


def kernel(z, emb):
    bs, dim, h, w = z.shape
    zp = jnp.transpose(z, (0, 2, 3, 1))
    z_flat = zp.reshape(-1, dim)
    zsq = jnp.sum(z_flat ** 2, axis=1, keepdims=True)
    esq = jnp.sum(emb ** 2, axis=1)[None, :]
    z_bf = z_flat.astype(jnp.bfloat16)
    emb_bf = emb.astype(jnp.bfloat16)
    idx, dsum = _tc_argmin(z_bf, emb_bf, zsq, esq)
    emb_r = emb_bf.astype(jnp.float32)
    z_q = _sc_gather_kernel()(emb_r, idx.reshape(-1))
    loss = jnp.sum(dsum) * (1.25 / (_N_TOK * _DIM))
    z_q_out = jnp.transpose(z_q.reshape(bs, h, w, dim), (0, 3, 1, 2))
    return (z_q_out, loss)
